# baseline (speedup 1.0000x reference)
import functools

import jax
import jax.numpy as jnp
from jax import lax
from jax.experimental import pallas as pl
from jax.experimental.pallas import tpu as pltpu
from jax.experimental.pallas import tpu_sc as plsc

NBUF = 8


@functools.cache
def _embed_call(B, T, V, D, NC, NS):
    NW = NC * NS
    TP = (T + 7) // 8 * 8
    rows_per_w = B // NW
    n_groups = rows_per_w // NBUF
    assert rows_per_w * NW == B and n_groups * NBUF == rows_per_w
    assert TP <= 128

    mesh = plsc.VectorSubcoreMesh(core_axis_name="c", subcore_axis_name="s")

    @functools.partial(
        pl.kernel,
        mesh=mesh,
        out_type=jax.ShapeDtypeStruct((B, T, D), jnp.float32),
        scratch_types=(
            [
                pltpu.VMEM((rows_per_w * TP,), jnp.int32),
                pltpu.VMEM((NBUF, TP, D), jnp.float32),
            ]
            + [pltpu.SemaphoreType.DMA] * (2 * NBUF)
        ),
        compiler_params=pltpu.CompilerParams(use_tc_tiling_on_sc=True),
    )
    def emb(idx_hbm, table_hbm, out_hbm, idx_v, rows_v, *sems):
        gsem = sems[:NBUF]
        ssem = sems[NBUF:]
        wid = lax.axis_index("s") * NC + lax.axis_index("c")
        base = wid * rows_per_w

        pltpu.sync_copy(idx_hbm.at[wid], idx_v)

        for b in range(NBUF):
            pltpu.async_copy(
                table_hbm.at[idx_v.at[pl.ds(b * TP, TP)]], rows_v.at[b], gsem[b]
            )

        def group(g, carry):
            for b in range(NBUF):
                c = g * NBUF + b
                bp = (b - 1) % NBUF

                @pl.when(c >= 1)
                def _refill():
                    pltpu.make_async_copy(
                        rows_v.at[bp].at[pl.ds(0, T)], out_hbm.at[0], ssem[bp]
                    ).wait()

                    @pl.when(c - 1 + NBUF < rows_per_w)
                    def _launch():
                        pltpu.async_copy(
                            table_hbm.at[idx_v.at[pl.ds((c - 1 + NBUF) * TP, TP)]],
                            rows_v.at[bp],
                            gsem[bp],
                        )

                pltpu.make_async_copy(
                    table_hbm.at[pl.ds(0, TP)], rows_v.at[b], gsem[b]
                ).wait()
                pltpu.async_copy(
                    rows_v.at[b].at[pl.ds(0, T)], out_hbm.at[base + c], ssem[b]
                )
            return carry

        lax.fori_loop(0, n_groups, group, 0)

        pltpu.make_async_copy(
            rows_v.at[NBUF - 1].at[pl.ds(0, T)], out_hbm.at[0], ssem[NBUF - 1]
        ).wait()

    return emb


def kernel(token_ids, weights):
    B, T = token_ids.shape
    V, D = weights.shape
    info = plsc.get_sparse_core_info()
    NC, NS = info.num_cores, info.num_subcores
    NW = NC * NS
    TP = (T + 7) // 8 * 8
    idx = jnp.pad(token_ids.astype(jnp.int32), ((0, 0), (0, TP - T)))
    idx = idx.reshape(NW, (B // NW) * TP)
    return _embed_call(B, T, V, D, NC, NS)(idx, weights)


# speedup vs baseline: 1.0027x; 1.0027x over previous
import functools

import jax
import jax.numpy as jnp
from jax import lax
from jax.experimental import pallas as pl
from jax.experimental.pallas import tpu as pltpu
from jax.experimental.pallas import tpu_sc as plsc

NBUF = 8


@functools.cache
def _embed_call(B, T, V, D, NC, NS):
    NW = NC * NS
    TP = (T + 7) // 8 * 8
    rows_per_w = B // NW
    n_groups = rows_per_w // NBUF
    assert rows_per_w * NW == B and n_groups * NBUF == rows_per_w
    assert TP <= 128

    mesh = plsc.VectorSubcoreMesh(core_axis_name="c", subcore_axis_name="s")

    @functools.partial(
        pl.kernel,
        mesh=mesh,
        out_type=jax.ShapeDtypeStruct((B, T, D), jnp.float32),
        scratch_types=(
            [
                pltpu.VMEM((rows_per_w, TP), jnp.int32),
                pltpu.VMEM((NBUF, TP, D), jnp.float32),
            ]
            + [pltpu.SemaphoreType.DMA] * (2 * NBUF)
        ),
        compiler_params=pltpu.CompilerParams(use_tc_tiling_on_sc=True),
    )
    def emb(idx_hbm, table_hbm, out_hbm, idx_v, rows_v, *sems):
        gsem = sems[:NBUF]
        ssem = sems[NBUF:]
        wid = lax.axis_index("s") * NC + lax.axis_index("c")
        base = wid * rows_per_w

        pltpu.sync_copy(idx_hbm.at[wid], idx_v)

        for b in range(NBUF):
            pltpu.async_copy(
                table_hbm.at[idx_v.at[b]], rows_v.at[b], gsem[b]
            )

        def group(g, carry):
            for b in range(NBUF):
                c = g * NBUF + b
                bp = (b - 1) % NBUF

                @pl.when(c >= 1)
                def _refill():
                    pltpu.make_async_copy(
                        rows_v.at[bp].at[pl.ds(0, T)], out_hbm.at[0], ssem[bp]
                    ).wait()

                    @pl.when(c - 1 + NBUF < rows_per_w)
                    def _launch():
                        pltpu.async_copy(
                            table_hbm.at[idx_v.at[c - 1 + NBUF]],
                            rows_v.at[bp],
                            gsem[bp],
                        )

                pltpu.make_async_copy(
                    table_hbm.at[pl.ds(0, TP)], rows_v.at[b], gsem[b]
                ).wait()
                pltpu.async_copy(
                    rows_v.at[b].at[pl.ds(0, T)], out_hbm.at[base + c], ssem[b]
                )
            return carry

        lax.fori_loop(0, n_groups, group, 0)

        pltpu.make_async_copy(
            rows_v.at[NBUF - 1].at[pl.ds(0, T)], out_hbm.at[0], ssem[NBUF - 1]
        ).wait()

    return emb


def kernel(token_ids, weights):
    B, T = token_ids.shape
    V, D = weights.shape
    info = plsc.get_sparse_core_info()
    NC, NS = info.num_cores, info.num_subcores
    NW = NC * NS
    TP = (T + 7) // 8 * 8
    idx = jnp.pad(token_ids.astype(jnp.int32), ((0, 0), (0, TP - T)))
    idx = idx.reshape(NW, B // NW, TP)
    return _embed_call(B, T, V, D, NC, NS)(idx, weights)
